# baseline (speedup 1.0000x reference)
import numpy as np
import jax
import jax.numpy as jnp
from jax.experimental import pallas as pl
from jax.experimental.pallas import tpu as pltpu

_N_COMP = 8
_TB = 512


def _body(x_ref, w_ref, b_ref, g_ref, mu_ref, std_ref):
    n = _N_COMP
    out = jnp.dot(x_ref[...], w_ref[...], preferred_element_type=jnp.float32)
    out = out + b_ref[...]
    dz = (out.shape[1] // n - 1) // 2
    mu_base = 0
    var_base = n * dz
    pi_base = 2 * n * dz

    pis = out[:, pi_base:pi_base + n]
    m = jnp.max(pis, axis=-1, keepdims=True)
    e = jnp.exp(pis - m)
    probs = e / jnp.sum(e, axis=-1, keepdims=True)

    s = jnp.log(probs + 1e-30) + g_ref[...]
    smax = jnp.max(s, axis=-1, keepdims=True)
    idx = jax.lax.broadcasted_iota(jnp.int32, s.shape, 1)
    k = jnp.min(jnp.where(s >= smax, idx, n), axis=-1, keepdims=True)

    lane = jax.lax.broadcasted_iota(jnp.int32, (out.shape[0], n * dz), 1)
    mask = (lane // dz) == k
    mu = jnp.where(mask, out[:, mu_base:mu_base + n * dz], 0.0)
    var = jnp.where(mask, out[:, var_base:var_base + n * dz], 0.0)
    w = n * dz
    while w > dz:
        w //= 2
        mu = mu[:, :w] + mu[:, w:]
        var = var[:, :w] + var[:, w:]

    std = jnp.sqrt(jax.nn.softplus(var) + 1e-08)
    std = jnp.clip(std, 1e-05, 100000.0)
    mu_ref[...] = mu
    std_ref[...] = std


def kernel(x, W, b):
    B, d_in = x.shape
    d_out = W.shape[1]
    n = _N_COMP
    dz = (d_out // n - 1) // 2
    span = 2 * dz + 1

    cols = np.arange(d_out).reshape(n, span)
    perm = np.concatenate([
        cols[:, :dz].reshape(-1),
        cols[:, dz:2 * dz].reshape(-1),
        cols[:, 2 * dz],
    ])
    Wp = W[:, perm]
    bp = b[perm].reshape(1, d_out)

    G = jax.random.gumbel(jax.random.key(42), (B, n), jnp.float32)

    grid = B // _TB
    mu, std = pl.pallas_call(
        _body,
        grid=(grid,),
        in_specs=[
            pl.BlockSpec((_TB, d_in), lambda i: (i, 0)),
            pl.BlockSpec((d_in, d_out), lambda i: (0, 0)),
            pl.BlockSpec((1, d_out), lambda i: (0, 0)),
            pl.BlockSpec((_TB, n), lambda i: (i, 0)),
        ],
        out_specs=[
            pl.BlockSpec((_TB, dz), lambda i: (i, 0)),
            pl.BlockSpec((_TB, dz), lambda i: (i, 0)),
        ],
        out_shape=[
            jax.ShapeDtypeStruct((B, dz), jnp.float32),
            jax.ShapeDtypeStruct((B, dz), jnp.float32),
        ],
        compiler_params=pltpu.CompilerParams(
            dimension_semantics=("parallel",),
        ),
    )(x, Wp, bp, G)
    return (mu, std)


# speedup vs baseline: 1.0787x; 1.0787x over previous
import numpy as np
import jax
import jax.numpy as jnp
from jax.experimental import pallas as pl
from jax.experimental.pallas import tpu as pltpu

_N_COMP = 8
_TB = 2048


def _body(x_ref, w_ref, b_ref, g_ref, mu_ref, std_ref):
    n = _N_COMP
    out = jnp.dot(x_ref[...], w_ref[...], preferred_element_type=jnp.float32)
    out = out + b_ref[...]
    dz = (out.shape[1] // n - 1) // 2
    mu_base = 0
    var_base = n * dz
    pi_base = 2 * n * dz

    pis = out[:, pi_base:pi_base + n]
    m = jnp.max(pis, axis=-1, keepdims=True)
    e = jnp.exp(pis - m)
    probs = e / jnp.sum(e, axis=-1, keepdims=True)

    s = jnp.log(probs + 1e-30) + g_ref[...]
    smax = jnp.max(s, axis=-1, keepdims=True)
    idx = jax.lax.broadcasted_iota(jnp.int32, s.shape, 1)
    k = jnp.min(jnp.where(s >= smax, idx, n), axis=-1, keepdims=True)

    lane = jax.lax.broadcasted_iota(jnp.int32, (out.shape[0], n * dz), 1)
    mask = (lane // dz) == k
    mu = jnp.where(mask, out[:, mu_base:mu_base + n * dz], 0.0)
    var = jnp.where(mask, out[:, var_base:var_base + n * dz], 0.0)
    w = n * dz
    while w > dz:
        w //= 2
        mu = mu[:, :w] + mu[:, w:]
        var = var[:, :w] + var[:, w:]

    std = jnp.sqrt(jax.nn.softplus(var) + 1e-08)
    std = jnp.clip(std, 1e-05, 100000.0)
    mu_ref[...] = mu
    std_ref[...] = std


def kernel(x, W, b):
    B, d_in = x.shape
    d_out = W.shape[1]
    n = _N_COMP
    dz = (d_out // n - 1) // 2
    span = 2 * dz + 1

    cols = np.arange(d_out).reshape(n, span)
    perm = np.concatenate([
        cols[:, :dz].reshape(-1),
        cols[:, dz:2 * dz].reshape(-1),
        cols[:, 2 * dz],
    ])
    Wp = W[:, perm]
    bp = b[perm].reshape(1, d_out)

    G = jax.random.gumbel(jax.random.key(42), (B, n), jnp.float32)

    grid = B // _TB
    mu, std = pl.pallas_call(
        _body,
        grid=(grid,),
        in_specs=[
            pl.BlockSpec((_TB, d_in), lambda i: (i, 0)),
            pl.BlockSpec((d_in, d_out), lambda i: (0, 0)),
            pl.BlockSpec((1, d_out), lambda i: (0, 0)),
            pl.BlockSpec((_TB, n), lambda i: (i, 0)),
        ],
        out_specs=[
            pl.BlockSpec((_TB, dz), lambda i: (i, 0)),
            pl.BlockSpec((_TB, dz), lambda i: (i, 0)),
        ],
        out_shape=[
            jax.ShapeDtypeStruct((B, dz), jnp.float32),
            jax.ShapeDtypeStruct((B, dz), jnp.float32),
        ],
        compiler_params=pltpu.CompilerParams(
            dimension_semantics=("parallel",),
        ),
    )(x, Wp, bp, G)
    return (mu, std)


# speedup vs baseline: 1.0797x; 1.0009x over previous
import numpy as np
import jax
import jax.numpy as jnp
from jax.experimental import pallas as pl
from jax.experimental.pallas import tpu as pltpu

_N_COMP = 8
_TB = 4096


def _body(x_ref, w_ref, b_ref, g_ref, mu_ref, std_ref):
    n = _N_COMP
    out = jnp.dot(x_ref[...], w_ref[...], preferred_element_type=jnp.float32)
    out = out + b_ref[...]
    dz = (out.shape[1] // n - 1) // 2
    mu_base = 0
    var_base = n * dz
    pi_base = 2 * n * dz

    pis = out[:, pi_base:pi_base + n]
    m = jnp.max(pis, axis=-1, keepdims=True)
    e = jnp.exp(pis - m)
    probs = e / jnp.sum(e, axis=-1, keepdims=True)

    s = jnp.log(probs + 1e-30) + g_ref[...]
    smax = jnp.max(s, axis=-1, keepdims=True)
    idx = jax.lax.broadcasted_iota(jnp.int32, s.shape, 1)
    k = jnp.min(jnp.where(s >= smax, idx, n), axis=-1, keepdims=True)

    lane = jax.lax.broadcasted_iota(jnp.int32, (out.shape[0], n * dz), 1)
    mask = (lane // dz) == k
    mu = jnp.where(mask, out[:, mu_base:mu_base + n * dz], 0.0)
    var = jnp.where(mask, out[:, var_base:var_base + n * dz], 0.0)
    w = n * dz
    while w > dz:
        w //= 2
        mu = mu[:, :w] + mu[:, w:]
        var = var[:, :w] + var[:, w:]

    std = jnp.sqrt(jax.nn.softplus(var) + 1e-08)
    std = jnp.clip(std, 1e-05, 100000.0)
    mu_ref[...] = mu
    std_ref[...] = std


def kernel(x, W, b):
    B, d_in = x.shape
    d_out = W.shape[1]
    n = _N_COMP
    dz = (d_out // n - 1) // 2
    span = 2 * dz + 1

    cols = np.arange(d_out).reshape(n, span)
    perm = np.concatenate([
        cols[:, :dz].reshape(-1),
        cols[:, dz:2 * dz].reshape(-1),
        cols[:, 2 * dz],
    ])
    Wp = W[:, perm]
    bp = b[perm].reshape(1, d_out)

    G = jax.random.gumbel(jax.random.key(42), (B, n), jnp.float32)

    grid = B // _TB
    mu, std = pl.pallas_call(
        _body,
        grid=(grid,),
        in_specs=[
            pl.BlockSpec((_TB, d_in), lambda i: (i, 0)),
            pl.BlockSpec((d_in, d_out), lambda i: (0, 0)),
            pl.BlockSpec((1, d_out), lambda i: (0, 0)),
            pl.BlockSpec((_TB, n), lambda i: (i, 0)),
        ],
        out_specs=[
            pl.BlockSpec((_TB, dz), lambda i: (i, 0)),
            pl.BlockSpec((_TB, dz), lambda i: (i, 0)),
        ],
        out_shape=[
            jax.ShapeDtypeStruct((B, dz), jnp.float32),
            jax.ShapeDtypeStruct((B, dz), jnp.float32),
        ],
        compiler_params=pltpu.CompilerParams(
            dimension_semantics=("parallel",),
        ),
    )(x, Wp, bp, G)
    return (mu, std)


# speedup vs baseline: 1.1344x; 1.0507x over previous
import numpy as np
import jax
import jax.numpy as jnp
from jax.experimental import pallas as pl
from jax.experimental.pallas import tpu as pltpu

_N_COMP = 8
_TB = 4096


def _body(x_ref, w_ref, b_ref, g_ref, mu_ref, std_ref):
    n = _N_COMP
    out = jnp.dot(x_ref[...], w_ref[...], preferred_element_type=jnp.float32)
    out = out + b_ref[...]
    dz = (out.shape[1] // n - 1) // 2
    mu_base = 0
    var_base = n * dz
    pi_base = 2 * n * dz

    pis = out[:, pi_base:pi_base + n]
    m = jnp.max(pis, axis=-1, keepdims=True)
    e = jnp.exp(pis - m)
    probs = e / jnp.sum(e, axis=-1, keepdims=True)

    s = jnp.log(probs + 1e-30) + g_ref[...]
    smax = jnp.max(s, axis=-1, keepdims=True)
    idx = jax.lax.broadcasted_iota(jnp.int32, s.shape, 1)
    k = jnp.min(jnp.where(s >= smax, idx, n), axis=-1, keepdims=True)

    lane = jax.lax.broadcasted_iota(jnp.int32, (out.shape[0], n * dz), 1)
    mask = (lane // dz) == k
    mu = jnp.where(mask, out[:, mu_base:mu_base + n * dz], 0.0)
    var = jnp.where(mask, out[:, var_base:var_base + n * dz], 0.0)
    w = n * dz
    while w > dz:
        w //= 2
        mu = mu[:, :w] + mu[:, w:]
        var = var[:, :w] + var[:, w:]

    std = jnp.sqrt(jax.nn.softplus(var) + 1e-08)
    std = jnp.clip(std, 1e-05, 100000.0)
    mu_ref[...] = mu
    std_ref[...] = std


def kernel(x, W, b):
    B, d_in = x.shape
    d_out = W.shape[1]
    n = _N_COMP
    dz = (d_out // n - 1) // 2
    span = 2 * dz + 1

    W3 = W.reshape(d_in, n, span)
    Wp = jnp.concatenate([
        W3[:, :, :dz].reshape(d_in, n * dz),
        W3[:, :, dz:2 * dz].reshape(d_in, n * dz),
        W3[:, :, 2 * dz],
    ], axis=1)
    b3 = b.reshape(n, span)
    bp = jnp.concatenate([
        b3[:, :dz].reshape(n * dz),
        b3[:, dz:2 * dz].reshape(n * dz),
        b3[:, 2 * dz],
    ]).reshape(1, d_out)

    G = jax.random.gumbel(jax.random.key(42), (B, n), jnp.float32)

    grid = B // _TB
    mu, std = pl.pallas_call(
        _body,
        grid=(grid,),
        in_specs=[
            pl.BlockSpec((_TB, d_in), lambda i: (i, 0)),
            pl.BlockSpec((d_in, d_out), lambda i: (0, 0)),
            pl.BlockSpec((1, d_out), lambda i: (0, 0)),
            pl.BlockSpec((_TB, n), lambda i: (i, 0)),
        ],
        out_specs=[
            pl.BlockSpec((_TB, dz), lambda i: (i, 0)),
            pl.BlockSpec((_TB, dz), lambda i: (i, 0)),
        ],
        out_shape=[
            jax.ShapeDtypeStruct((B, dz), jnp.float32),
            jax.ShapeDtypeStruct((B, dz), jnp.float32),
        ],
        compiler_params=pltpu.CompilerParams(
            dimension_semantics=("parallel",),
        ),
    )(x, Wp, bp, G)
    return (mu, std)
